# baseline (speedup 1.0000x reference)
import jax
import jax.numpy as jnp
from jax import lax
from jax.experimental import pallas as pl
from jax.experimental.pallas import tpu as pltpu
from jax.experimental.pallas import tpu_sc as plsc

N = 10000
E = 320000
C = 128
L = 2

NC = 2
NS = 16
NPAD = 10240
ROWS_PER_SUB = NPAD // NS
EDGES_PER_SUB = E // (NC * NS)
CHUNK = 80
NCHUNK = EDGES_PER_SUB // CHUNK
NBUF = 3
BATCH = 25
NBATCH = NCHUNK // BATCH


def _segsum_body(h_hbm, pidx_hbm, out_hbm,
                 acc, pring, uidx, rows,
                 isem, gsem0, gsem1, gsem2, ssem0, ssem1, ssem2):
    gsems = (gsem0, gsem1, gsem2)
    ssems = (ssem0, ssem1, ssem2)
    cid = lax.axis_index("c")
    sid = lax.axis_index("s")
    wid = cid * NS + sid

    def refill(r):
        pltpu.async_copy(pidx_hbm.at[wid, r], pring.at[r % 3], isem)

    def refill_wait(r):
        pltpu.make_async_copy(pidx_hbm.at[wid, r], pring.at[r % 3],
                              isem).wait()

    refill(0)

    @pl.loop(0, CHUNK)
    def _(r):
        @pl.loop(0, C, step=16)
        def _(c):
            rows[2, r, pl.ds(c, 16)] = jnp.zeros((16,), jnp.float32)

    refill_wait(0)
    refill(1)

    def unpack_at(slot, loc, u):
        @pl.loop(0, CHUNK, step=16)
        def _(c):
            p = pring[slot, loc, pl.ds(c, 16)]
            uidx[2 * u, pl.ds(c, 16)] = lax.bitwise_and(p, 0xFFFF)
            uidx[2 * u + 1, pl.ds(c, 16)] = lax.shift_right_logical(p, 16)

    def unpack(k):
        unpack_at(k // BATCH % 3, k % BATCH, k % 4)

    def fire(k):
        b, u = k % 3, k % 4
        pltpu.async_copy(h_hbm.at[uidx.at[2 * u]], rows.at[b], gsems[b])

    def step(k, tail=False):
        b, u = k % 3, k % 4
        pltpu.make_async_copy(h_hbm.at[uidx.at[2 * u]], rows.at[b],
                              gsems[b]).wait()
        pltpu.async_copy(rows.at[b], acc.at[uidx.at[2 * u + 1]], ssems[b],
                         add=True)
        if not tail:
            unpack(k + NBUF)
        pltpu.make_async_copy(rows.at[b], acc.at[uidx.at[2 * u + 1]],
                              ssems[b]).wait()
        if not tail:
            fire(k + NBUF)

    unpack(0)
    unpack(1)
    unpack(2)
    fire(0)
    fire(1)

    row0 = sid * ROWS_PER_SUB

    @pl.loop(0, ROWS_PER_SUB, step=CHUNK)
    def _(r):
        pltpu.sync_copy(rows.at[2], acc.at[pl.ds(row0 + r, CHUNK)])

    fire(2)
    plsc.subcore_barrier()

    for r in range(NBATCH):
        base = r * BATCH
        last = r == NBATCH - 1
        slot = r % 3

        @pl.loop(0, 21, step=3)
        def _(j, base=base, slot=slot):
            for s in range(3):
                b = (base + s) % 3
                u2 = 2 * jnp.bitwise_and(j + (base + s), 3)
                u3 = 2 * jnp.bitwise_and(j + (base + s) + NBUF, 3)
                pltpu.make_async_copy(h_hbm.at[uidx.at[u2]], rows.at[b],
                                      gsems[b]).wait()
                pltpu.async_copy(rows.at[b], acc.at[uidx.at[u2 + 1]],
                                 ssems[b], add=True)
                unpack_at(slot, j + s + NBUF, jnp.bitwise_and(
                    j + (base + s) + NBUF, 3))
                pltpu.make_async_copy(rows.at[b], acc.at[uidx.at[u2 + 1]],
                                      ssems[b]).wait()
                pltpu.async_copy(h_hbm.at[uidx.at[u3]], rows.at[b],
                                 gsems[b])

        step(base + 21)
        if not last:
            refill_wait(r + 1)
            if r + 2 < NBATCH:
                refill(r + 2)
            for c in (22, 23, 24):
                step(base + c)
        else:
            for c in (22, 23, 24):
                step(base + c, tail=True)

    plsc.subcore_barrier()
    pltpu.sync_copy(acc.at[pl.ds(row0, ROWS_PER_SUB)],
                    out_hbm.at[cid, pl.ds(row0, ROWS_PER_SUB)])


def _segment_sum_sc(h, pidx):
    mesh = plsc.VectorSubcoreMesh(core_axis_name="c", subcore_axis_name="s",
                                  num_cores=NC, num_subcores=NS)
    kern = pl.kernel(
        _segsum_body,
        out_type=jax.ShapeDtypeStruct((NC, NPAD, C), jnp.float32),
        mesh=mesh,
        scratch_types=[
            pltpu.VMEM_SHARED((NPAD, C), jnp.float32),
            pltpu.VMEM((3, BATCH, CHUNK), jnp.int32),
            pltpu.VMEM((8, CHUNK), jnp.int32),
            pltpu.VMEM((NBUF, CHUNK, C), jnp.float32),
        ] + [pltpu.SemaphoreType.DMA] * 7,
    )
    return kern(h, pidx)


def _pack_edges(src, dst):
    packed = jnp.bitwise_or(src, jnp.left_shift(dst, 16))
    return packed.reshape(NC * NS, NBATCH, BATCH, CHUNK)


BR = 2048


def _gru_body(p_ref, h_ref, w_ref, wih_ref, whh_ref, bih_ref, bhh_ref, out_ref):
    agg = p_ref[0] + p_ref[1]
    aggw = jnp.dot(agg, w_ref[...], preferred_element_type=jnp.float32)
    gi = jnp.dot(aggw, wih_ref[...], preferred_element_type=jnp.float32)
    gi = gi + bih_ref[...]
    h = h_ref[...]
    gh = jnp.dot(h, whh_ref[...], preferred_element_type=jnp.float32)
    gh = gh + bhh_ref[...]
    r = jax.nn.sigmoid(gi[:, :C] + gh[:, :C])
    z = jax.nn.sigmoid(gi[:, C:2 * C] + gh[:, C:2 * C])
    n = jnp.tanh(gi[:, 2 * C:] + r * gh[:, 2 * C:])
    out_ref[...] = (1.0 - z) * n + z * h


def _gru_tc(p, h, w, w_ihT, w_hhT, b_ih2, b_hh2):
    grid = (NPAD // BR,)
    return pl.pallas_call(
        _gru_body,
        grid=grid,
        in_specs=[
            pl.BlockSpec((NC, BR, C), lambda i: (0, i, 0)),
            pl.BlockSpec((BR, C), lambda i: (i, 0)),
            pl.BlockSpec((C, C), lambda i: (0, 0)),
            pl.BlockSpec((C, 3 * C), lambda i: (0, 0)),
            pl.BlockSpec((C, 3 * C), lambda i: (0, 0)),
            pl.BlockSpec((1, 3 * C), lambda i: (0, 0)),
            pl.BlockSpec((1, 3 * C), lambda i: (0, 0)),
        ],
        out_specs=pl.BlockSpec((BR, C), lambda i: (i, 0)),
        out_shape=jax.ShapeDtypeStruct((N, C), jnp.float32),
    )(p, h, w, w_ihT, w_hhT, b_ih2, b_hh2)


def kernel(x, edge_index, weight, w_ih, w_hh, b_ih, b_hh):
    pidx = _pack_edges(edge_index[0], edge_index[1])
    w_ihT = w_ih.T
    w_hhT = w_hh.T
    b_ih2 = b_ih.reshape(1, 3 * C)
    b_hh2 = b_hh.reshape(1, 3 * C)
    h = x
    for i in range(L):
        p = _segment_sum_sc(h, pidx)
        h = _gru_tc(p, h, weight[i], w_ihT, w_hhT, b_ih2, b_hh2)
    return h


# speedup vs baseline: 1.0050x; 1.0050x over previous
import jax
import jax.numpy as jnp
from jax import lax
from jax.experimental import pallas as pl
from jax.experimental.pallas import tpu as pltpu
from jax.experimental.pallas import tpu_sc as plsc

N = 10000
E = 320000
C = 128
L = 2

NC = 2
NS = 16
NPAD = 10240
ROWS_PER_SUB = NPAD // NS
EDGES_PER_SUB = E // (NC * NS)
CHUNK = 80
NCHUNK = EDGES_PER_SUB // CHUNK
NBUF = 3
BATCH = 25
NBATCH = NCHUNK // BATCH


def _segsum_body(h_hbm, pidx_hbm, out_hbm,
                 acc, pring, uidx, rows,
                 isem, gsem0, gsem1, gsem2, ssem0, ssem1, ssem2):
    gsems = (gsem0, gsem1, gsem2)
    ssems = (ssem0, ssem1, ssem2)
    cid = lax.axis_index("c")
    sid = lax.axis_index("s")
    wid = cid * NS + sid

    def refill(r):
        pltpu.async_copy(pidx_hbm.at[wid, r], pring.at[r % 3], isem)

    def refill_wait(r):
        pltpu.make_async_copy(pidx_hbm.at[wid, r], pring.at[r % 3],
                              isem).wait()

    refill(0)

    @pl.loop(0, CHUNK)
    def _(r):
        @pl.loop(0, C, step=16)
        def _(c):
            rows[2, r, pl.ds(c, 16)] = jnp.zeros((16,), jnp.float32)

    refill_wait(0)
    refill(1)

    def unpack_at(slot, loc, u):
        @pl.loop(0, CHUNK, step=16)
        def _(c):
            p = pring[slot, loc, pl.ds(c, 16)]
            uidx[2 * u, pl.ds(c, 16)] = lax.bitwise_and(p, 0xFFFF)
            uidx[2 * u + 1, pl.ds(c, 16)] = lax.shift_right_logical(p, 16)

    def unpack(k):
        unpack_at(k // BATCH % 3, k % BATCH, k % 4)

    def fire(k):
        b, u = k % 3, k % 4
        pltpu.async_copy(h_hbm.at[uidx.at[2 * u]], rows.at[b], gsems[b])

    def step(k, tail=False):
        b, u = k % 3, k % 4
        pltpu.make_async_copy(h_hbm.at[uidx.at[2 * u]], rows.at[b],
                              gsems[b]).wait()
        pltpu.async_copy(rows.at[b], acc.at[uidx.at[2 * u + 1]], ssems[b],
                         add=True)
        if not tail:
            unpack(k + NBUF)
        pltpu.make_async_copy(rows.at[b], acc.at[uidx.at[2 * u + 1]],
                              ssems[b]).wait()
        if not tail:
            fire(k + NBUF)

    unpack(0)
    unpack(1)
    unpack(2)
    fire(0)
    fire(1)

    row0 = sid * ROWS_PER_SUB

    @pl.loop(0, ROWS_PER_SUB, step=CHUNK)
    def _(r):
        pltpu.sync_copy(rows.at[2], acc.at[pl.ds(row0 + r, CHUNK)])

    fire(2)
    plsc.subcore_barrier()

    for r in range(NBATCH):
        base = r * BATCH
        last = r == NBATCH - 1
        slot = r % 3

        @pl.loop(0, 21, step=3)
        def _(j, base=base, slot=slot):
            for s in range(3):
                b = (base + s) % 3
                u2 = 2 * jnp.bitwise_and(j + (base + s), 3)
                u3 = 2 * jnp.bitwise_and(j + (base + s) + NBUF, 3)
                pltpu.make_async_copy(h_hbm.at[uidx.at[u2]], rows.at[b],
                                      gsems[b]).wait()
                pltpu.async_copy(rows.at[b], acc.at[uidx.at[u2 + 1]],
                                 ssems[b], add=True)
                unpack_at(slot, j + s + NBUF, jnp.bitwise_and(
                    j + (base + s) + NBUF, 3))
                pltpu.make_async_copy(rows.at[b], acc.at[uidx.at[u2 + 1]],
                                      ssems[b]).wait()
                pltpu.async_copy(h_hbm.at[uidx.at[u3]], rows.at[b],
                                 gsems[b])

        step(base + 21)
        if not last:
            refill_wait(r + 1)
            if r + 2 < NBATCH:
                refill(r + 2)
            for c in (22, 23, 24):
                step(base + c)
        else:
            for c in (22, 23, 24):
                step(base + c, tail=True)

    plsc.subcore_barrier()
    pltpu.sync_copy(acc.at[pl.ds(row0, ROWS_PER_SUB)],
                    out_hbm.at[cid, pl.ds(row0, ROWS_PER_SUB)])


def _segment_sum_sc(h, pidx):
    mesh = plsc.VectorSubcoreMesh(core_axis_name="c", subcore_axis_name="s",
                                  num_cores=NC, num_subcores=NS)
    kern = pl.kernel(
        _segsum_body,
        out_type=jax.ShapeDtypeStruct((NC, NPAD, C), jnp.float32),
        mesh=mesh,
        scratch_types=[
            pltpu.VMEM_SHARED((NPAD, C), jnp.float32),
            pltpu.VMEM((3, BATCH, CHUNK), jnp.int32),
            pltpu.VMEM((8, CHUNK), jnp.int32),
            pltpu.VMEM((NBUF, CHUNK, C), jnp.float32),
        ] + [pltpu.SemaphoreType.DMA] * 7,
    )
    return kern(h, pidx)


def _pack_edges(src, dst):
    packed = jnp.bitwise_or(src, jnp.left_shift(dst, 16))
    return packed.reshape(NC * NS, NBATCH, BATCH, CHUNK)


BR = 2560


def _gru_body(p_ref, h_ref, w_ref, wih_ref, whh_ref, bih_ref, bhh_ref, out_ref):
    agg = p_ref[0] + p_ref[1]
    aggw = jnp.dot(agg, w_ref[...], preferred_element_type=jnp.float32)
    gi = jnp.dot(aggw, wih_ref[...], preferred_element_type=jnp.float32)
    gi = gi + bih_ref[...]
    h = h_ref[...]
    gh = jnp.dot(h, whh_ref[...], preferred_element_type=jnp.float32)
    gh = gh + bhh_ref[...]
    r = jax.nn.sigmoid(gi[:, :C] + gh[:, :C])
    z = jax.nn.sigmoid(gi[:, C:2 * C] + gh[:, C:2 * C])
    n = jnp.tanh(gi[:, 2 * C:] + r * gh[:, 2 * C:])
    out_ref[...] = (1.0 - z) * n + z * h


def _gru_tc(p, h, w, w_ihT, w_hhT, b_ih2, b_hh2):
    grid = (NPAD // BR,)
    return pl.pallas_call(
        _gru_body,
        grid=grid,
        in_specs=[
            pl.BlockSpec((NC, BR, C), lambda i: (0, i, 0)),
            pl.BlockSpec((BR, C), lambda i: (i, 0)),
            pl.BlockSpec((C, C), lambda i: (0, 0)),
            pl.BlockSpec((C, 3 * C), lambda i: (0, 0)),
            pl.BlockSpec((C, 3 * C), lambda i: (0, 0)),
            pl.BlockSpec((1, 3 * C), lambda i: (0, 0)),
            pl.BlockSpec((1, 3 * C), lambda i: (0, 0)),
        ],
        out_specs=pl.BlockSpec((BR, C), lambda i: (i, 0)),
        out_shape=jax.ShapeDtypeStruct((N, C), jnp.float32),
    )(p, h, w, w_ihT, w_hhT, b_ih2, b_hh2)


def kernel(x, edge_index, weight, w_ih, w_hh, b_ih, b_hh):
    pidx = _pack_edges(edge_index[0], edge_index[1])
    w_ihT = w_ih.T
    w_hhT = w_hh.T
    b_ih2 = b_ih.reshape(1, 3 * C)
    b_hh2 = b_hh.reshape(1, 3 * C)
    h = x
    for i in range(L):
        p = _segment_sum_sc(h, pidx)
        h = _gru_tc(p, h, weight[i], w_ihT, w_hhT, b_ih2, b_hh2)
    return h
